# baseline (speedup 1.0000x reference)
import functools

import jax
import jax.numpy as jnp
from jax import lax
from jax.experimental import pallas as pl
from jax.experimental.pallas import tpu as pltpu
from jax.experimental.pallas import tpu_sc as plsc


def kernel(x, pos_table):
    B, T, D = x.shape
    NW = 32
    ROWS = B * T
    RPW = ROWS // NW
    R = 32
    NSTEPS = RPW // R
    CH = R * D

    x_flat = x.reshape(ROWS * D)
    pos_flat = pos_table.reshape(-1)

    mesh = plsc.VectorSubcoreMesh(core_axis_name="c", subcore_axis_name="s")

    @functools.partial(
        pl.kernel,
        mesh=mesh,
        out_type=jax.ShapeDtypeStruct((ROWS * D,), jnp.float32),
        scratch_types=[
            pltpu.VMEM((CH,), jnp.float32),
            pltpu.VMEM((CH,), jnp.float32),
        ],
    )
    def sc_add(x_hbm, pos_hbm, out_hbm, x_buf, pos_buf):
        c = lax.axis_index("c")
        s = lax.axis_index("s")
        wid = c * 16 + s
        row0 = wid * RPW
        prow0 = lax.rem(row0, T)

        def step(k, carry):
            base = pl.multiple_of((row0 + k * R) * D, 1024)
            pbase = pl.multiple_of((prow0 + k * R) * D, 1024)
            pltpu.sync_copy(x_hbm.at[pl.ds(base, CH)], x_buf)
            pltpu.sync_copy(pos_hbm.at[pl.ds(pbase, CH)], pos_buf)

            def add_body(i, carry2):
                sl = pl.ds(pl.multiple_of(i * 16, 16), 16)
                pos_buf[sl] = pos_buf[sl] + x_buf[sl]
                return carry2

            lax.fori_loop(0, CH // 16, add_body, 0)
            pltpu.sync_copy(pos_buf, out_hbm.at[pl.ds(base, CH)])
            return carry

        lax.fori_loop(0, NSTEPS, step, 0)

    out = sc_add(x_flat, pos_flat)
    return out.reshape(B, T, D)


# speedup vs baseline: 1.4726x; 1.4726x over previous
import functools

import jax
import jax.numpy as jnp
from jax import lax
from jax.experimental import pallas as pl
from jax.experimental.pallas import tpu as pltpu
from jax.experimental.pallas import tpu_sc as plsc


def kernel(x, pos_table):
    B, T, D = x.shape
    NW = 32
    ROWS = B * T
    RPW = ROWS // NW
    R = 16
    NSTEPS = RPW // R
    CH = R * D
    NB = 2

    x_flat = x.reshape(ROWS * D)
    pos_flat = pos_table.reshape(-1)

    mesh = plsc.VectorSubcoreMesh(core_axis_name="c", subcore_axis_name="s")

    @functools.partial(
        pl.kernel,
        mesh=mesh,
        out_type=jax.ShapeDtypeStruct((ROWS * D,), jnp.float32),
        scratch_types=[
            pltpu.VMEM((NB, CH), jnp.float32),
            pltpu.VMEM((NB, CH), jnp.float32),
            pltpu.SemaphoreType.DMA((NB,)),
            pltpu.SemaphoreType.DMA((NB,)),
            pltpu.SemaphoreType.DMA((NB,)),
        ],
    )
    def sc_add(x_hbm, pos_hbm, out_hbm, x_buf, pos_buf, xsem, psem, osem):
        c = lax.axis_index("c")
        s = lax.axis_index("s")
        wid = c * 16 + s
        row0 = wid * RPW
        prow0 = lax.rem(row0, T)

        def xbase(k):
            return pl.multiple_of((row0 + k * R) * D, 1024)

        def pbase(k):
            return pl.multiple_of((prow0 + k * R) * D, 1024)

        def start_loads(k):
            p = k % NB
            dx = pltpu.async_copy(
                x_hbm.at[pl.ds(xbase(k), CH)], x_buf.at[p], xsem.at[p])
            dp = pltpu.async_copy(
                pos_hbm.at[pl.ds(pbase(k), CH)], pos_buf.at[p], psem.at[p])
            return dx, dp

        loads = {0: start_loads(0)}
        stores = {}
        for k in range(NSTEPS):
            p = k % NB
            if k + 1 < NSTEPS:
                if k - 1 in stores:
                    stores.pop(k - 1).wait()
                loads[k + 1] = start_loads(k + 1)
            dx, dp = loads.pop(k)
            dx.wait()
            dp.wait()

            @plsc.parallel_loop(0, CH, step=16, unroll=8)
            def _(i):
                sl = pl.ds(pl.multiple_of(i, 16), 16)
                plsc.addupdate(pos_buf.at[p].at[sl], x_buf[p, sl])

            stores[k] = pltpu.async_copy(
                pos_buf.at[p], out_hbm.at[pl.ds(xbase(k), CH)], osem.at[p])
        for k in sorted(stores):
            stores.pop(k).wait()

    out = sc_add(x_flat, pos_flat)
    return out.reshape(B, T, D)


# speedup vs baseline: 4.1950x; 2.8488x over previous
import functools

import jax
import jax.numpy as jnp
from jax import lax
from jax.experimental import pallas as pl
from jax.experimental.pallas import tpu as pltpu
from jax.experimental.pallas import tpu_sc as plsc


def kernel(x, pos_table):
    B, T, D = x.shape
    NW = 32
    ROWS = B * T
    RPW = ROWS // NW
    R = 16
    NSTEPS = RPW // R
    NB = 2

    x_flat = x.reshape(ROWS, D)

    mesh = plsc.VectorSubcoreMesh(core_axis_name="c", subcore_axis_name="s")

    @functools.partial(
        pl.kernel,
        mesh=mesh,
        out_type=jax.ShapeDtypeStruct((ROWS, D), jnp.float32),
        scratch_types=[
            pltpu.VMEM((NB, R, D), jnp.float32),
            pltpu.VMEM((NB, R, D), jnp.float32),
            pltpu.SemaphoreType.DMA((NB,)),
            pltpu.SemaphoreType.DMA((NB,)),
            pltpu.SemaphoreType.DMA((NB,)),
        ],
    )
    def sc_add(x_hbm, pos_hbm, out_hbm, x_buf, pos_buf, xsem, psem, osem):
        c = lax.axis_index("c")
        s = lax.axis_index("s")
        wid = c * 16 + s
        row0 = wid * RPW
        prow0 = lax.rem(row0, T)

        def xrow(k):
            return pl.multiple_of(row0 + k * R, R)

        def prow(k):
            return pl.multiple_of(prow0 + k * R, R)

        def start_loads(k):
            p = k % NB
            dx = pltpu.async_copy(
                x_hbm.at[pl.ds(xrow(k), R)], x_buf.at[p], xsem.at[p])
            dp = pltpu.async_copy(
                pos_hbm.at[pl.ds(prow(k), R)], pos_buf.at[p], psem.at[p])
            return dx, dp

        loads = {0: start_loads(0)}
        stores = {}
        for k in range(NSTEPS):
            p = k % NB
            if k + 1 < NSTEPS:
                if k - 1 in stores:
                    stores.pop(k - 1).wait()
                loads[k + 1] = start_loads(k + 1)
            dx, dp = loads.pop(k)
            dx.wait()
            dp.wait()

            @plsc.parallel_loop(0, R * D, step=16, unroll=8)
            def _(i):
                r = i // D
                d0 = pl.multiple_of(i % D, 16)
                sl = pl.ds(d0, 16)
                plsc.addupdate(pos_buf.at[p, r].at[sl], x_buf[p, r, sl])

            stores[k] = pltpu.async_copy(
                pos_buf.at[p], out_hbm.at[pl.ds(xrow(k), R)], osem.at[p])
        for k in sorted(stores):
            stores.pop(k).wait()

    out = sc_add(x_flat, pos_table)
    return out.reshape(B, T, D)
